# baseline (speedup 1.0000x reference)
import jax
import jax.numpy as jnp
from jax import lax
from jax.experimental import pallas as pl
from jax.experimental.pallas import tpu as pltpu
from jax.experimental.pallas import tpu_sc as plsc

P = 32
M = 512
N = 512
IMG = M * N
NP = N + 2
PROWS = 544
IMGP = PROWS * NP
ROWS_PER_CHUNK = 8
C = ROWS_PER_CHUNK * N
CU = 2 * C
NUM_CHUNKS = IMG // C
VECS = C // 16
BLKW = ROWS_PER_CHUNK * NP
NSLOT = 8
WINW = NSLOT * BLKW


def _warp_body(xp_hbm, u_hbm, out_hbm,
               win, ub0, ub1, ob0, ob1,
               f00, f01, f10, f11,
               g00b, g01b, g10b, g11b,
               sem_u, sem_pref, sem_out0, sem_out1, sem_g):
    cid = lax.axis_index("c")
    sid = lax.axis_index("s")
    wid = sid * 2 + cid

    pbase = (wid // 8) * IMGP
    ubase = wid * IMG * 2
    obase = wid * IMG

    lanef = lax.broadcasted_iota(jnp.int32, (16,), 0).astype(jnp.float32)
    iota2 = lax.broadcasted_iota(jnp.int32, (16,), 0) * 2

    def coords(ub, t, k):
        offu = t * 32
        dxl = ub[pl.ds(offu, 16)]
        dyl = ub[pl.ds(offu + 16, 16)]
        jxf = ((t & 31) * 16 + 1).astype(jnp.float32)
        iyf = (k * 8 + (t >> 5) + 1).astype(jnp.float32)
        xs = dxl + lanef + jxf
        ys = dyl + iyf
        xs = jnp.minimum(jnp.maximum(xs, 0.0), float(NP - 1))
        ys = jnp.minimum(jnp.maximum(ys, 0.0), float(NP - 1))
        x0 = xs.astype(jnp.int32)
        y0 = ys.astype(jnp.int32)
        wx = xs - x0.astype(jnp.float32)
        wy = ys - y0.astype(jnp.float32)
        return x0, y0, wx, wy

    def do_chunk(k, ub_cur, ub_nxt, ob_cur, sem_out):
        pltpu.make_async_copy(
            u_hbm.at[pl.ds(ubase + k * CU, CU)], ub_cur, sem_u).wait()

        @pl.when(k < NUM_CHUNKS - 1)
        def _():
            pltpu.async_copy(
                u_hbm.at[pl.ds(ubase + (k + 1) * CU, CU)], ub_nxt, sem_u)

        @pl.when(k > NUM_CHUNKS)
        def _():
            pltpu.make_async_copy(
                xp_hbm.at[pl.ds(pbase, BLKW)], win.at[pl.ds(0, BLKW)],
                sem_pref).wait()

            blk = k + 3
            slot = blk & (NSLOT - 1)
            pltpu.async_copy(
                xp_hbm.at[pl.ds(pbase + blk * BLKW, BLKW)],
                win.at[pl.ds(slot * BLKW, BLKW)], sem_pref)

        @pl.when(k >= 2)
        def _():
            pltpu.make_async_copy(
                ob_cur, out_hbm.at[pl.ds(obase, C)], sem_out).wait()

        wlo = k * 8 - 32
        whi = k * 8 + 24

        def vec_body(t, missv):
            off = t * 16
            ob_cur[pl.ds(off, 16)] = ub_cur[pl.ds(off, 16)]
            return missv

        def vec_body_unused(t, missv):
            off = t * 16
            x0, y0, wx, wy = coords(ub_cur, t, k)
            y1 = y0 + 1
            x1 = x0 + 1

            in0 = (y0 >= wlo) & (y0 < whi)
            in1 = (y1 >= wlo) & (y1 < whi)
            miss = ~(in0 & in1)

            lb0 = (y0 & (NSLOT * 8 - 1)) * NP
            lb1 = (y1 & (NSLOT * 8 - 1)) * NP
            v00 = lb0.astype(jnp.float32)
            v01 = lb1.astype(jnp.float32)
            v10 = x0.astype(jnp.float32)
            v11 = x1.astype(jnp.float32)

            ox = 1.0 - wx
            oy = 1.0 - wy
            acc = oy * (v00 * ox + v01 * wx) + wy * (v10 * ox + v11 * wx)
            ob_cur[pl.ds(off, 16)] = acc
            return missv | miss.astype(jnp.int32)

        missv = lax.fori_loop(0, VECS, vec_body,
                              jnp.zeros((16,), jnp.int32))
        nmiss = jnp.max(missv)

        @pl.when(nmiss > 2 ** 30)
        def _fallback():
            def idx_body(t, carry):
                off = t * 16
                x0, y0, _wx, _wy = coords(ub_cur, t, k)
                yb0 = y0 * NP + pbase
                yb1 = yb0 + NP
                f00[pl.ds(off, 16)] = yb0 + x0
                f01[pl.ds(off, 16)] = yb0 + x0 + 1
                f10[pl.ds(off, 16)] = yb1 + x0
                f11[pl.ds(off, 16)] = yb1 + x0 + 1
                return carry

            lax.fori_loop(0, VECS, idx_body, None)

            c0 = pltpu.async_copy(xp_hbm.at[f00], g00b, sem_g)
            c1 = pltpu.async_copy(xp_hbm.at[f01], g01b, sem_g)
            c2 = pltpu.async_copy(xp_hbm.at[f10], g10b, sem_g)
            c3 = pltpu.async_copy(xp_hbm.at[f11], g11b, sem_g)
            c0.wait()
            c1.wait()
            c2.wait()
            c3.wait()

            def mix_body(t, carry):
                off = t * 16
                _x0, _y0, wx, wy = coords(ub_cur, t, k)
                ox = 1.0 - wx
                oy = 1.0 - wy
                s = pl.ds(off, 16)
                acc = (oy * (g00b[s] * ox + g01b[s] * wx)
                       + wy * (g10b[s] * ox + g11b[s] * wx))
                ob_cur[s] = acc
                return carry

            lax.fori_loop(0, VECS, mix_body, None)

        pltpu.async_copy(ob_cur, out_hbm.at[pl.ds(obase + k * C, C)],
                         sem_out)

    pltpu.async_copy(u_hbm.at[pl.ds(ubase, CU)], ub0, sem_u)
    pltpu.sync_copy(xp_hbm.at[pl.ds(pbase, 3 * BLKW)],
                    win.at[pl.ds(0, 3 * BLKW)])
    win[pl.ds(WINW, 16)] = jnp.zeros((16,), jnp.float32)

    def pair_body(m, _):
        do_chunk(2 * m, ub0, ub1, ob0, sem_out0)
        do_chunk(2 * m + 1, ub1, ub0, ob1, sem_out1)
        return _

    lax.fori_loop(0, NUM_CHUNKS // 2, pair_body, None)

    pltpu.make_async_copy(ob0, out_hbm.at[pl.ds(obase, C)], sem_out0).wait()
    pltpu.make_async_copy(ob1, out_hbm.at[pl.ds(obase, C)], sem_out1).wait()


@jax.jit
def _warp_call(xp_flat, u_flat):
    mesh = plsc.VectorSubcoreMesh(core_axis_name="c", subcore_axis_name="s")
    f = pl.kernel(
        _warp_body,
        out_type=jax.ShapeDtypeStruct((P * IMG,), jnp.float32),
        mesh=mesh,
        compiler_params=pltpu.CompilerParams(needs_layout_passes=False),
        scratch_types=[
            pltpu.VMEM((WINW + 16,), jnp.float32),
            pltpu.VMEM((CU,), jnp.float32),
            pltpu.VMEM((CU,), jnp.float32),
            pltpu.VMEM((C,), jnp.float32),
            pltpu.VMEM((C,), jnp.float32),
            pltpu.VMEM((C,), jnp.int32),
            pltpu.VMEM((C,), jnp.int32),
            pltpu.VMEM((C,), jnp.int32),
            pltpu.VMEM((C,), jnp.int32),
            pltpu.VMEM((C,), jnp.float32),
            pltpu.VMEM((C,), jnp.float32),
            pltpu.VMEM((C,), jnp.float32),
            pltpu.VMEM((C,), jnp.float32),
            pltpu.SemaphoreType.DMA,
            pltpu.SemaphoreType.DMA,
            pltpu.SemaphoreType.DMA,
            pltpu.SemaphoreType.DMA,
            pltpu.SemaphoreType.DMA,
        ],
    )
    return f(xp_flat, u_flat)


def kernel(x, u):
    xp = jnp.zeros((4, PROWS, NP), jnp.float32)
    xp = xp.at[:, 1:M + 1, 1:N + 1].set(x)
    out = _warp_call(xp.reshape(-1), u.reshape(-1))
    return out.reshape(u.shape[:-1])


# speedup vs baseline: 13.5639x; 13.5639x over previous
import functools

import jax
import jax.numpy as jnp
from jax import lax
from jax.experimental import pallas as pl
from jax.experimental.pallas import tpu as pltpu
from jax.experimental.pallas import tpu_sc as plsc

P = 32
M = 512
N = 512
IMG = M * N
ROWS_PER_CHUNK = 8
C = ROWS_PER_CHUNK * N
NUM_CHUNKS = IMG // C
VECS = C // 16


def _floor_parts(v):
    ti = v.astype(jnp.int32)
    tf = ti.astype(jnp.float32)
    fi = jnp.where(tf > v, ti - 1, ti)
    w = v - fi.astype(jnp.float32)
    return fi, w


def _warp_body(x_hbm, dx_hbm, dy_hbm, out_hbm,
               img_s, dxv, dyv,
               i00, i01, i10, i11,
               w00, w01, w10, w11,
               v00, v01, v10, v11,
               outv, sem_in, sem_g, sem_out):
    nc = 2
    cid = lax.axis_index("c")
    sid = lax.axis_index("s")
    wid = sid * nc + cid

    @pl.when(sid == 0)
    def _stage():
        pltpu.sync_copy(x_hbm, img_s)

    plsc.subcore_barrier()

    pbase = (wid // 8) * IMG
    ubase = wid * IMG

    lane = lax.broadcasted_iota(jnp.int32, (16,), 0).astype(jnp.float32)

    def chunk_body(ck, _):
        base = ubase + ck * C
        cpx = pltpu.async_copy(dx_hbm.at[pl.ds(base, C)], dxv, sem_in)
        cpy = pltpu.async_copy(dy_hbm.at[pl.ds(base, C)], dyv, sem_in)
        cpx.wait()
        cpy.wait()

        r0 = ck * ROWS_PER_CHUNK

        def vec_body(t, _):
            off = t * 16
            iy = (r0 + off // N).astype(jnp.float32)
            jx = (off % N).astype(jnp.float32)

            dxl = dxv[pl.ds(off, 16)]
            dyl = dyv[pl.ds(off, 16)]
            xs = lane + jx + dxl
            ys = iy + dyl
            xs = jnp.minimum(jnp.maximum(xs, -2.0), 513.0)
            ys = jnp.minimum(jnp.maximum(ys, -2.0), 513.0)

            x0, wx = _floor_parts(xs)
            y0, wy = _floor_parts(ys)

            vx0 = (x0 >= 0) & (x0 <= N - 1)
            vx1 = (x0 >= -1) & (x0 <= N - 2)
            vy0 = (y0 >= 0) & (y0 <= M - 1)
            vy1 = (y0 >= -1) & (y0 <= M - 2)

            x0c = jnp.minimum(jnp.maximum(x0, 0), N - 1)
            x1c = jnp.minimum(jnp.maximum(x0 + 1, 0), N - 1)
            y0c = jnp.minimum(jnp.maximum(y0, 0), M - 1)
            y1c = jnp.minimum(jnp.maximum(y0 + 1, 0), M - 1)

            yb0 = y0c * N + pbase
            yb1 = y1c * N + pbase
            i00[pl.ds(off, 16)] = yb0 + x0c
            i01[pl.ds(off, 16)] = yb0 + x1c
            i10[pl.ds(off, 16)] = yb1 + x0c
            i11[pl.ds(off, 16)] = yb1 + x1c

            ox = 1.0 - wx
            oy = 1.0 - wy
            zero = jnp.zeros((16,), jnp.float32)
            w00[pl.ds(off, 16)] = jnp.where(vx0 & vy0, ox * oy, zero)
            w01[pl.ds(off, 16)] = jnp.where(vx1 & vy0, wx * oy, zero)
            w10[pl.ds(off, 16)] = jnp.where(vx0 & vy1, ox * wy, zero)
            w11[pl.ds(off, 16)] = jnp.where(vx1 & vy1, wx * wy, zero)
            return _

        lax.fori_loop(0, VECS, vec_body, None)

        g0 = pltpu.async_copy(img_s.at[i00], v00, sem_g)
        g1 = pltpu.async_copy(img_s.at[i01], v01, sem_g)
        g2 = pltpu.async_copy(img_s.at[i10], v10, sem_g)
        g3 = pltpu.async_copy(img_s.at[i11], v11, sem_g)
        g0.wait()
        g1.wait()
        g2.wait()
        g3.wait()

        def mix_body(t, _):
            off = t * 16
            acc = (w00[pl.ds(off, 16)] * v00[pl.ds(off, 16)]
                   + w01[pl.ds(off, 16)] * v01[pl.ds(off, 16)]
                   + w10[pl.ds(off, 16)] * v10[pl.ds(off, 16)]
                   + w11[pl.ds(off, 16)] * v11[pl.ds(off, 16)])
            outv[pl.ds(off, 16)] = acc
            return _

        lax.fori_loop(0, VECS, mix_body, None)

        pltpu.sync_copy(outv, out_hbm.at[pl.ds(base, C)])
        return _

    lax.fori_loop(0, NUM_CHUNKS, chunk_body, None)


@jax.jit
def _warp_call(x_flat, dx_flat, dy_flat):
    mesh = plsc.VectorSubcoreMesh(core_axis_name="c", subcore_axis_name="s")
    f = pl.kernel(
        _warp_body,
        out_type=jax.ShapeDtypeStruct((P * IMG,), jnp.float32),
        mesh=mesh,
        compiler_params=pltpu.CompilerParams(needs_layout_passes=False),
        scratch_types=[
            pltpu.VMEM_SHARED((4 * IMG,), jnp.float32),
            pltpu.VMEM((C,), jnp.float32),
            pltpu.VMEM((C,), jnp.float32),
            pltpu.VMEM((C,), jnp.int32),
            pltpu.VMEM((C,), jnp.int32),
            pltpu.VMEM((C,), jnp.int32),
            pltpu.VMEM((C,), jnp.int32),
            pltpu.VMEM((C,), jnp.float32),
            pltpu.VMEM((C,), jnp.float32),
            pltpu.VMEM((C,), jnp.float32),
            pltpu.VMEM((C,), jnp.float32),
            pltpu.VMEM((C,), jnp.float32),
            pltpu.VMEM((C,), jnp.float32),
            pltpu.VMEM((C,), jnp.float32),
            pltpu.VMEM((C,), jnp.float32),
            pltpu.VMEM((C,), jnp.float32),
            pltpu.SemaphoreType.DMA,
            pltpu.SemaphoreType.DMA,
            pltpu.SemaphoreType.DMA,
        ],
    )
    return f(x_flat, dx_flat, dy_flat)


def kernel(x, u):
    x_flat = x.reshape(-1)
    dx_flat = u[..., 0].reshape(-1)
    dy_flat = u[..., 1].reshape(-1)
    out = _warp_call(x_flat, dx_flat, dy_flat)
    return out.reshape(u.shape[:-1])
